# Initial kernel scaffold; baseline (speedup 1.0000x reference)
import jax
import jax.numpy as jnp
from jax.experimental import pallas as pl


def kernel(x, edge_index, edge_attr, W1, att_src1, att_dst1, We1, att_e1, b1, W2, att_src2, att_dst2, We2, att_e2, b2, a_prelu):
    raise NotImplementedError("write your pallas kernel here")


# speedup vs baseline: 17.7547x; 17.7547x over previous
import functools

import jax
import jax.numpy as jnp
from jax import lax
from jax.experimental import pallas as pl
from jax.experimental.pallas import tpu as pltpu
from jax.experimental.pallas import tpu_sc as plsc

_N = 10000
_E = 320000
_D = 128
_NC = 2
_NS = 16
_NW = _NC * _NS
_NPAD = 10240
_C = 10368
_EPAD = _NW * _C
_C16 = _C // 16
_DK = _C // 128
_GK = 64
_GCH = _C // _GK
_RPT = _NPAD // _NS

_GDN = lax.GatherDimensionNumbers(
    offset_dims=(), collapsed_slice_dims=(0,), start_index_map=(0,))


def _lane_bcast(v16, k):
    idx = jnp.full((16, 1), k, dtype=jnp.int32)
    return lax.gather(v16, idx, _GDN, (1,),
                      mode=lax.GatherScatterMode.PROMISE_IN_BOUNDS)


def _tc_layer1_body(x_ref, w_ref, asw_ref, adw_ref, we_ref, ate_ref, ea_ref,
                    h_ref, as_ref, ad_ref, cst_ref):
    h = jnp.dot(x_ref[...], w_ref[...], preferred_element_type=jnp.float32)
    h_ref[...] = h
    a_s = jnp.dot(h, asw_ref[...].reshape(_D, 1),
                  preferred_element_type=jnp.float32)
    a_d = jnp.dot(h, adw_ref[...].reshape(_D, 1),
                  preferred_element_type=jnp.float32)
    as_ref[...] = a_s
    ad_ref[...] = a_d
    c = jnp.sum(we_ref[...] * ate_ref[...])
    ea = ea_ref[...]
    mean_ea = jnp.sum(ea) * (1.0 / _E)
    max_ea = jnp.max(ea)
    min_ea = jnp.min(ea)
    max_ae = jnp.maximum(jnp.maximum(c * max_ea, c * min_ea), c * mean_ea)
    bb = jnp.maximum(jnp.max(a_s) + jnp.max(a_d) + max_ae, 0.0)
    cst_ref[...] = jnp.stack(
        [jnp.full((_D,), v, jnp.float32)
         for v in (c, bb, mean_ea, max_ea, min_ea)])


_tc_layer1 = pl.pallas_call(
    _tc_layer1_body,
    out_shape=[
        jax.ShapeDtypeStruct((_NPAD, _D), jnp.float32),
        jax.ShapeDtypeStruct((_NPAD, 1), jnp.float32),
        jax.ShapeDtypeStruct((_NPAD, 1), jnp.float32),
        jax.ShapeDtypeStruct((5, _D), jnp.float32),
    ],
)


def _tc_layer2_body(parts_ref, b_ref, ap_ref, w_ref, asw_ref, adw_ref,
                    we_ref, ate_ref, cst1_ref,
                    h_ref, as_ref, ad_ref, cst_ref):
    z = parts_ref[0] + parts_ref[1] + b_ref[...]
    ap = ap_ref[0, 0]
    xin = jnp.where(z >= 0.0, z, ap * z)
    h = jnp.dot(xin, w_ref[...], preferred_element_type=jnp.float32)
    h_ref[...] = h
    a_s = jnp.dot(h, asw_ref[...].reshape(_D, 1),
                  preferred_element_type=jnp.float32)
    a_d = jnp.dot(h, adw_ref[...].reshape(_D, 1),
                  preferred_element_type=jnp.float32)
    as_ref[...] = a_s
    ad_ref[...] = a_d
    c = jnp.sum(we_ref[...] * ate_ref[...])
    mean_ea = cst1_ref[2, 0]
    max_ea = cst1_ref[3, 0]
    min_ea = cst1_ref[4, 0]
    max_ae = jnp.maximum(jnp.maximum(c * max_ea, c * min_ea), c * mean_ea)
    bb = jnp.maximum(jnp.max(a_s) + jnp.max(a_d) + max_ae, 0.0)
    cst_ref[...] = jnp.stack(
        [jnp.full((_D,), v, jnp.float32)
         for v in (c, bb, mean_ea, max_ea, min_ea)])


_tc_layer2 = pl.pallas_call(
    _tc_layer2_body,
    out_shape=[
        jax.ShapeDtypeStruct((_NPAD, _D), jnp.float32),
        jax.ShapeDtypeStruct((_NPAD, 1), jnp.float32),
        jax.ShapeDtypeStruct((_NPAD, 1), jnp.float32),
        jax.ShapeDtypeStruct((5, _D), jnp.float32),
    ],
)


def _tc_dsum_body(d_ref, o_ref):
    o_ref[...] = d_ref[0:1, :] + d_ref[1:2, :]


_tc_dsum = pl.pallas_call(
    _tc_dsum_body,
    out_shape=jax.ShapeDtypeStruct((1, _NPAD), jnp.float32),
)


def _tc_final_body(parts_ref, b_ref, ap_ref, y_ref):
    z = parts_ref[0] + parts_ref[1] + b_ref[...]
    ap = ap_ref[0, 0]
    y_ref[...] = jnp.where(z >= 0.0, z, ap * z)


_tc_final = pl.pallas_call(
    _tc_final_body,
    out_shape=jax.ShapeDtypeStruct((_NPAD, _D), jnp.float32),
)


_MESH = plsc.VectorSubcoreMesh(core_axis_name="c", subcore_axis_name="s")


@functools.partial(
    pl.kernel,
    mesh=_MESH,
    compiler_params=pltpu.CompilerParams(needs_layout_passes=False),
    out_type=[
        jax.ShapeDtypeStruct((_NW, _C), jnp.float32),
        jax.ShapeDtypeStruct((_NC, _NPAD), jnp.float32),
    ],
    scratch_types=[
        pltpu.VMEM((_C,), jnp.int32),
        pltpu.VMEM((_C,), jnp.int32),
        pltpu.VMEM((_DK, 128), jnp.int32),
        pltpu.VMEM((_C,), jnp.float32),
        pltpu.VMEM((_NPAD,), jnp.float32),
        pltpu.VMEM((_NPAD,), jnp.float32),
        pltpu.VMEM((640,), jnp.float32),
        pltpu.VMEM((_C,), jnp.float32),
        pltpu.VMEM((_RPT,), jnp.float32),
        pltpu.VMEM_SHARED((_NPAD,), jnp.float32),
    ],
)
def _sc_pass_a(src_h, dstf_h, dstm_h, ea_h, asrc_h, adst_h, consts_h,
               ex_h, den_h,
               src_v, dstf_v, dstm_v, ea_v, asrc_v, adst_v, consts_v,
               ex_v, z_v, den_sh):
    cid = lax.axis_index("c")
    sid = lax.axis_index("s")
    wid = cid * _NS + sid
    pltpu.sync_copy(src_h.at[wid], src_v)
    pltpu.sync_copy(dstf_h.at[wid], dstf_v)
    pltpu.sync_copy(dstm_h.at[wid], dstm_v)
    pltpu.sync_copy(ea_h.at[wid], ea_v)
    pltpu.sync_copy(asrc_h, asrc_v)
    pltpu.sync_copy(adst_h, adst_v)
    pltpu.sync_copy(consts_h, consts_v)

    zero = jnp.zeros((16,), jnp.float32)

    def zfill(i, carry):
        z_v[pl.ds(i * 16, 16)] = zero
        return carry

    lax.fori_loop(0, _RPT // 16, zfill, 0)
    pltpu.sync_copy(z_v, den_sh.at[pl.ds(sid * _RPT, _RPT)])
    plsc.subcore_barrier()

    c16 = consts_v[pl.ds(0, 16)]
    b16 = consts_v[pl.ds(128, 16)]

    def chunk(i, carry):
        s16 = src_v[pl.ds(i * 16, 16)]
        d16 = dstf_v[pl.ds(i * 16, 16)]
        ae = ea_v[pl.ds(i * 16, 16)] * c16
        a = (plsc.load_gather(asrc_v, [s16])
             + plsc.load_gather(adst_v, [d16]) + ae)
        a = jnp.where(a >= 0.0, a, 0.2 * a)
        ex_v[pl.ds(i * 16, 16)] = jnp.exp(a - b16)
        return carry

    lax.fori_loop(0, _C16, chunk, 0)

    def dchunk(j, carry):
        pltpu.sync_copy(ex_v.at[pl.ds(j * 128, 128)],
                        den_sh.at[dstm_v.at[j]], add=True)
        return carry

    lax.fori_loop(0, _DK, dchunk, 0)
    pltpu.sync_copy(ex_v, ex_h.at[wid])
    plsc.subcore_barrier()
    pltpu.sync_copy(den_sh.at[pl.ds(sid * _RPT, _RPT)],
                    den_h.at[cid, pl.ds(sid * _RPT, _RPT)])


@functools.partial(
    pl.kernel,
    mesh=_MESH,
    compiler_params=pltpu.CompilerParams(needs_layout_passes=False),
    out_type=jax.ShapeDtypeStruct((_NC, _NPAD, _D), jnp.float32),
    scratch_types=[
        pltpu.VMEM((_GK,), jnp.int32),
        pltpu.VMEM((_GK,), jnp.int32),
        pltpu.VMEM((_GK,), jnp.float32),
        pltpu.VMEM((_NPAD,), jnp.float32),
        pltpu.VMEM((_GK, _D), jnp.float32),
        pltpu.VMEM((16, _D), jnp.float32),
        pltpu.VMEM_SHARED((_NPAD, _D), jnp.float32),
        pltpu.SemaphoreType.DMA,
    ],
)
def _sc_pass_b(srcg_h, dstg_h, exg_h, den_h, h_h,
               out_h,
               srcc_v, dstc_v, exc_v, den_v, rows_v, zb_v, out_sh, sem):
    cid = lax.axis_index("c")
    sid = lax.axis_index("s")
    wid = cid * _NS + sid
    pltpu.sync_copy(den_h, den_v)

    zero = jnp.zeros((16,), jnp.float32)
    for r in range(16):
        for j in range(8):
            zb_v[r, pl.ds(j * 16, 16)] = zero
    r0 = sid * _RPT

    def zloop(t, carry):
        pltpu.sync_copy(zb_v, out_sh.at[pl.ds(r0 + t * 16, 16)])
        return carry

    lax.fori_loop(0, _RPT // 16, zloop, 0)
    plsc.subcore_barrier()

    def gloop(g, carry):
        pltpu.sync_copy(srcg_h.at[wid, g], srcc_v)
        pltpu.sync_copy(dstg_h.at[wid, g], dstc_v)
        pltpu.sync_copy(exg_h.at[wid, g], exc_v)
        pltpu.async_copy(h_h.at[srcc_v], rows_v, sem).wait()
        for kg in range(_GK // 16):
            d16 = plsc.load_gather(den_v, [dstc_v[pl.ds(kg * 16, 16)]])
            c16 = exc_v[pl.ds(kg * 16, 16)] / (d16 + 1e-16)
            for k in range(16):
                ck = _lane_bcast(c16, k)
                row = kg * 16 + k
                for j in range(8):
                    rows_v[row, pl.ds(j * 16, 16)] = (
                        rows_v[row, pl.ds(j * 16, 16)] * ck)
        pltpu.sync_copy(rows_v, out_sh.at[dstc_v], add=True)
        return carry

    lax.fori_loop(0, _GCH, gloop, 0)
    plsc.subcore_barrier()
    pltpu.sync_copy(out_sh.at[pl.ds(r0, _RPT)],
                    out_h.at[cid, pl.ds(r0, _RPT)])


def kernel(x, edge_index, edge_attr, W1, att_src1, att_dst1, We1, att_e1, b1,
           W2, att_src2, att_dst2, We2, att_e2, b2, a_prelu):
    f32 = jnp.float32
    x_pad = jnp.pad(x, ((0, _NPAD - _N), (0, 0)))
    ea_mat = edge_attr.reshape(_E // _D, _D)

    h1, asrc1, adst1, consts1 = _tc_layer1(
        x_pad, W1, att_src1.reshape(1, _D), att_dst1.reshape(1, _D),
        We1, att_e1.reshape(1, _D), ea_mat)

    pad_e = _EPAD - _E - _N
    loop = jnp.arange(_N, dtype=jnp.int32)
    src = jnp.concatenate(
        [edge_index[0].astype(jnp.int32), loop, jnp.zeros((pad_e,), jnp.int32)])
    dst = jnp.concatenate(
        [edge_index[1].astype(jnp.int32), loop,
         jnp.full((pad_e,), _N, jnp.int32)])
    mean_ea = consts1[2, 0]
    ea_ext = jnp.concatenate(
        [edge_attr[:, 0], jnp.full((_N,), mean_ea, f32),
         jnp.zeros((pad_e,), f32)])

    src_flat = src.reshape(_NW, _C)
    dst_flat = dst.reshape(_NW, _C)
    dst_mat = dst.reshape(_NW, _DK, 128)
    src_g = src.reshape(_NW, _GCH, _GK)
    dst_g = dst.reshape(_NW, _GCH, _GK)
    ea_flat = ea_ext.reshape(_NW, _C)

    def consts_flat(c):
        return jnp.pad(c.reshape(5 * _D), (0, 640 - 5 * _D))

    ex1, den1 = _sc_pass_a(src_flat, dst_flat, dst_mat, ea_flat,
                           asrc1.reshape(_NPAD), adst1.reshape(_NPAD),
                           consts_flat(consts1))
    dtot1 = _tc_dsum(den1).reshape(_NPAD)
    parts1 = _sc_pass_b(src_g, dst_g, ex1.reshape(_NW, _GCH, _GK),
                        dtot1, h1)

    h2, asrc2, adst2, consts2 = _tc_layer2(
        parts1, b1.reshape(1, _D), a_prelu.reshape(1, 1), W2,
        att_src2.reshape(1, _D), att_dst2.reshape(1, _D), We2,
        att_e2.reshape(1, _D), consts1)

    ex2, den2 = _sc_pass_a(src_flat, dst_flat, dst_mat, ea_flat,
                           asrc2.reshape(_NPAD), adst2.reshape(_NPAD),
                           consts_flat(consts2))
    dtot2 = _tc_dsum(den2).reshape(_NPAD)
    parts2 = _sc_pass_b(src_g, dst_g, ex2.reshape(_NW, _GCH, _GK),
                        dtot2, h2)

    y = _tc_final(parts2, b2.reshape(1, _D), a_prelu.reshape(1, 1))
    return y[:_N]
